# baseline (speedup 1.0000x reference)
import functools

import jax
import jax.numpy as jnp
from jax import lax
from jax.experimental import pallas as pl
from jax.experimental.pallas import tpu as pltpu
from jax.experimental.pallas import tpu_sc as plsc

_D = 64
_G = 64


@functools.partial(jax.jit, static_argnames=("num_rows",))
def _sc_gather(weight, idx_grouped, num_rows):
    info = plsc.get_sparse_core_info()
    nw = info.num_cores * info.num_subcores
    rows_per_w = num_rows // nw
    n_groups = rows_per_w // _G
    mesh = plsc.VectorSubcoreMesh(core_axis_name="c", subcore_axis_name="s")

    @functools.partial(
        pl.kernel,
        mesh=mesh,
        out_type=jax.ShapeDtypeStruct((num_rows, _D), jnp.float32),
        compiler_params=pltpu.CompilerParams(use_tc_tiling_on_sc=False),
        scratch_types=[
            pltpu.VMEM((n_groups, _G), jnp.int32),
            pltpu.VMEM((4, _G, _D), jnp.float32),
            pltpu.SemaphoreType.DMA((4,)),
            pltpu.SemaphoreType.DMA,
        ],
    )
    def k(table_hbm, idx_hbm, out_hbm, idx_v, rows_v, gsem, ssem):
        wid = lax.axis_index("s") * info.num_cores + lax.axis_index("c")
        base = wid * rows_per_w
        pltpu.sync_copy(
            idx_hbm.at[pl.ds(pl.multiple_of(wid * n_groups, 8), n_groups)], idx_v
        )

        def fire(m, bk):
            pltpu.async_copy(
                table_hbm.at[idx_v.at[m]], rows_v.at[bk], gsem.at[bk]
            )

        def wait_gather(bk):
            pltpu.make_async_copy(
                table_hbm.at[pl.ds(0, _G)], rows_v.at[bk], gsem.at[bk]
            ).wait()

        for bk in range(4):
            fire(bk, bk)

        @pl.loop(0, n_groups, step=4)
        def _ring(i):
            for bk in range(4):
                m = i + bk
                wait_gather(bk)

                @pl.when(m + 4 < n_groups)
                def _():
                    fire(m + 4, bk)

        pltpu.async_copy(
            rows_v.at[0], out_hbm.at[pl.ds(pl.multiple_of(base, _G), _G)], ssem
        )
        pltpu.make_async_copy(
            table_hbm.at[pl.ds(0, _G)], rows_v.at[0], ssem
        ).wait()

    return k(weight, idx_grouped)


def kernel(token_ids, weight):
    b, s = token_ids.shape
    num_rows = b * s
    idx_grouped = token_ids.astype(jnp.int32).reshape(num_rows // _G, _G)
    out = _sc_gather(weight, idx_grouped, num_rows)
    return out.reshape(b, s, _D)


# speedup vs baseline: 1.1898x; 1.1898x over previous
import functools

import jax
import jax.numpy as jnp
from jax import lax
from jax.experimental import pallas as pl
from jax.experimental.pallas import tpu as pltpu
from jax.experimental.pallas import tpu_sc as plsc

_G = 128
_NB = 4


@functools.partial(jax.jit, static_argnames=("num_rows",))
def _sc_gather(table128, idx_grouped, num_rows):
    info = plsc.get_sparse_core_info()
    nw = info.num_cores * info.num_subcores
    rows_per_w = num_rows // nw
    n_groups = rows_per_w // _G
    mesh = plsc.VectorSubcoreMesh(core_axis_name="c", subcore_axis_name="s")

    @functools.partial(
        pl.kernel,
        mesh=mesh,
        out_type=jax.ShapeDtypeStruct((num_rows, 128), jnp.float32),
        compiler_params=pltpu.CompilerParams(use_tc_tiling_on_sc=False),
        scratch_types=[
            pltpu.VMEM((n_groups, _G), jnp.int32),
            pltpu.VMEM((_NB, _G, 128), jnp.float32),
            pltpu.SemaphoreType.DMA((_NB,)),
            pltpu.SemaphoreType.DMA((_NB,)),
        ],
    )
    def k(table_hbm, idx_hbm, out_hbm, idx_v, rows_v, gsem, ssem):
        wid = lax.axis_index("s") * info.num_cores + lax.axis_index("c")
        base = wid * rows_per_w
        pltpu.sync_copy(
            idx_hbm.at[pl.ds(pl.multiple_of(wid * n_groups, 8), n_groups)], idx_v
        )

        def fire(m, bk):
            pltpu.async_copy(table_hbm.at[idx_v.at[m]], rows_v.at[bk], gsem.at[bk])

        def wait_gather(bk):
            pltpu.make_async_copy(
                table_hbm.at[pl.ds(0, _G)], rows_v.at[bk], gsem.at[bk]
            ).wait()

        def store(m, bk):
            off = pl.multiple_of(base + m * _G, _G)
            pltpu.async_copy(rows_v.at[bk], out_hbm.at[pl.ds(off, _G)], ssem.at[bk])

        def wait_store(bk):
            pltpu.make_async_copy(
                table_hbm.at[pl.ds(0, _G)], rows_v.at[bk], ssem.at[bk]
            ).wait()

        for bk in range(_NB):
            fire(bk, bk)

        @pl.loop(0, n_groups, step=_NB)
        def _ring(i):
            for bk in range(_NB):
                m = i + bk
                wait_gather(bk)
                store(m, bk)

                @pl.when(m + _NB < n_groups)
                def _():
                    wait_store(bk)
                    fire(m + _NB, bk)

        for bk in range(_NB):
            wait_store(bk)

    return k(table128, idx_grouped)


def kernel(token_ids, weight):
    b, s = token_ids.shape
    v, d = weight.shape
    num_rows = b * s
    table128 = jnp.pad(weight, ((0, 0), (0, 128 - d)))
    idx_grouped = token_ids.astype(jnp.int32).reshape(num_rows // _G, _G)
    out128 = _sc_gather(table128, idx_grouped, num_rows)
    return out128[:, :d].reshape(b, s, d)


# speedup vs baseline: 1.2862x; 1.0810x over previous
import functools

import jax
import jax.numpy as jnp
from jax import lax
from jax.experimental import pallas as pl
from jax.experimental.pallas import tpu as pltpu
from jax.experimental.pallas import tpu_sc as plsc

_G = 128
_NB = 4


@functools.partial(jax.jit, static_argnames=("num_rows",))
def _sc_gather(table128, idx_grouped, num_rows):
    info = plsc.get_sparse_core_info()
    nw = info.num_cores * info.num_subcores
    rows_per_w = num_rows // nw
    n_groups = rows_per_w // _G
    mesh = plsc.VectorSubcoreMesh(core_axis_name="c", subcore_axis_name="s")

    @functools.partial(
        pl.kernel,
        mesh=mesh,
        out_type=jax.ShapeDtypeStruct((num_rows, 128), jnp.float32),
        compiler_params=pltpu.CompilerParams(use_tc_tiling_on_sc=False),
        scratch_types=[
            pltpu.VMEM((n_groups, _G), jnp.int32),
            pltpu.VMEM((_NB, _G, 128), jnp.float32),
            pltpu.SemaphoreType.DMA((_NB,)),
            pltpu.SemaphoreType.DMA((_NB,)),
        ],
    )
    def k(table_hbm, idx_hbm, out_hbm, idx_v, rows_v, gsem, ssem):
        wid = lax.axis_index("s") * info.num_cores + lax.axis_index("c")
        base = wid * rows_per_w
        pltpu.sync_copy(
            idx_hbm.at[pl.ds(pl.multiple_of(wid * n_groups, 8), n_groups)], idx_v
        )

        def fire(m, bk):
            pltpu.async_copy(table_hbm.at[idx_v.at[m]], rows_v.at[bk], gsem.at[bk])

        def wait_gather(bk):
            pltpu.make_async_copy(
                table_hbm.at[pl.ds(0, _G)], rows_v.at[bk], gsem.at[bk]
            ).wait()

        def store(m, bk):
            off = pl.multiple_of(base + m * _G, _G)
            pltpu.async_copy(
                rows_v.at[bk, :, pl.ds(0, 64)],
                out_hbm.at[pl.ds(off, _G), pl.ds(0, 64)],
                ssem.at[bk],
            )

        def wait_store(bk):
            pltpu.make_async_copy(
                table_hbm.at[pl.ds(0, _G), pl.ds(0, 64)],
                rows_v.at[bk, :, pl.ds(0, 64)],
                ssem.at[bk],
            ).wait()

        for bk in range(_NB):
            fire(bk, bk)

        @pl.loop(0, n_groups, step=_NB)
        def _ring(i):
            for bk in range(_NB):
                m = i + bk
                wait_gather(bk)
                store(m, bk)

                @pl.when(m + _NB < n_groups)
                def _():
                    wait_store(bk)
                    fire(m + _NB, bk)

        for bk in range(_NB):
            wait_store(bk)

    return k(table128, idx_grouped)


def kernel(token_ids, weight):
    b, s = token_ids.shape
    v, d = weight.shape
    num_rows = b * s
    table128 = jnp.pad(weight, ((0, 0), (0, 128 - d)))
    idx_grouped = token_ids.astype(jnp.int32).reshape(num_rows // _G, _G)
    out128 = _sc_gather(table128, idx_grouped, num_rows)
    return out128[:, :d].reshape(b, s, d)
